# baseline (speedup 1.0000x reference)
import functools

import jax
import jax.numpy as jnp
import numpy as np
from jax import lax
from jax.experimental import pallas as pl
from jax.experimental.pallas import tpu as pltpu
from jax.experimental.pallas import tpu_sc as plsc

B = 8
GH = GW = 64
NBOX = 5
N = GH * GW * NBOX
NR, NL = 160, 128
MAX_OUT = 100
IOU_T = 0.4
NEG = np.float32(-1e30)
L = 16
STEPS = N // L


SH = 4
NSH = N // SH
SSTEPS = NSH // L


def _nms_tile_body(x1h, y1h, x2h, y2h, sch, outh, mailh,
                   x1v, y1v, x2v, y2v, livev, outv, candv,
                   r0v, r1v, r2v, r3v, sem0, sem1, sem2, sem3):
    cid = lax.axis_index("c")
    sid = lax.axis_index("s")
    img = cid * 4 + sid // SH
    shard = sid % SH
    base = shard * NSH
    row = cid * 16 + sid

    pltpu.sync_copy(x1h.at[row], x1v)
    pltpu.sync_copy(y1h.at[row], y1v)
    pltpu.sync_copy(x2h.at[row], x2v)
    pltpu.sync_copy(y2h.at[row], y2v)
    pltpu.sync_copy(sch.at[row], livev)

    iota = lax.iota(jnp.int32, L)
    zeros16 = jnp.zeros((L,), jnp.float32)
    big = jnp.float32(3e38)
    bigv = jnp.full((L,), big, jnp.float32)
    zerov = jnp.zeros((L,), jnp.float32)

    @plsc.parallel_loop(0, SSTEPS, unroll=8)
    def _zero(s):
        outv[pl.ds(s * L, L)] = zeros16

    def local_pass(best_loc, bx1, by1, bx2, by2, barea):
        init = (jnp.full((L,), -3e38, jnp.float32),
                jnp.full((L,), NSH, jnp.int32))

        @plsc.parallel_loop(0, SSTEPS, unroll=16, carry=init)
        def scan(s, carry):
            rmax, ridx = carry
            sl = pl.ds(s * L, L)
            x1s = x1v[sl]
            y1s = y1v[sl]
            x2s = x2v[sl]
            y2s = y2v[sl]
            lv = livev[sl]
            iw = jnp.maximum(jnp.minimum(x2s, bx2) - jnp.maximum(x1s, bx1), 0.0)
            ih = jnp.maximum(jnp.minimum(y2s, by2) - jnp.maximum(y1s, by1), 0.0)
            inter = iw * ih
            areas = (x2s - x1s) * (y2s - y1s)
            iou = inter / jnp.maximum(areas + barea - inter, 1e-9)
            idxs = iota + s * L
            supp = (iou > IOU_T) | (idxs == best_loc)
            nl = jnp.where(supp, NEG, lv)
            livev[sl] = nl
            upd = (nl > rmax) | ((nl == rmax) & (idxs < ridx))
            rmax = jnp.where(upd, nl, rmax)
            ridx = jnp.where(upd, idxs, ridx)
            return rmax, ridx

        rmax, ridx = scan
        gm = jnp.max(rmax)
        gi = jnp.min(jnp.where(rmax == gm, ridx, jnp.int32(NSH)))
        return gm, gi

    gm_l, gi_l = local_pass(jnp.int32(-1), bigv, bigv, bigv, bigv, zerov)

    def round_body(r, carry):
        gm_l, gi_l = carry
        par = r % 2
        rtag = r.astype(jnp.float32) + 0.25
        gsplat = jnp.zeros((L,), jnp.int32) + gi_l
        cx1 = plsc.load_gather(x1v, [gsplat])
        cy1 = plsc.load_gather(y1v, [gsplat])
        cx2 = plsc.load_gather(x2v, [gsplat])
        cy2 = plsc.load_gather(y2v, [gsplat])
        cand = jnp.where(iota == 0, zerov + gm_l,
               jnp.where(iota == 1, zerov + (gi_l + base).astype(jnp.float32),
               jnp.where(iota == 2, cx1,
               jnp.where(iota == 3, cy1,
               jnp.where(iota == 4, cx2,
               jnp.where(iota == 5, cy2, zerov + rtag))))))
        candv[...] = cand
        slot = par * 32 + img * SH
        pltpu.sync_copy(candv, mailh.at[slot + shard])

        def read_rows():
            c0 = pltpu.make_async_copy(mailh.at[slot + 0], r0v, sem0)
            c1 = pltpu.make_async_copy(mailh.at[slot + 1], r1v, sem1)
            c2 = pltpu.make_async_copy(mailh.at[slot + 2], r2v, sem2)
            c3 = pltpu.make_async_copy(mailh.at[slot + 3], r3v, sem3)
            c0.start(); c1.start(); c2.start(); c3.start()
            c0.wait(); c1.wait(); c2.wait(); c3.wait()
            return r0v[...], r1v[...], r2v[...], r3v[...]

        def stale(rows):
            return ((rows[0][6] != rtag) | (rows[1][6] != rtag)
                    | (rows[2][6] != rtag) | (rows[3][6] != rtag))

        rows0 = read_rows()
        rows = lax.while_loop(stale, lambda _: read_rows(), rows0)
        r0 = rows[0]
        gm, gi_f = r0[0], r0[1]
        bx1s, by1s, bx2s, by2s = r0[2], r0[3], r0[4], r0[5]
        for j in range(1, SH):
            rj = rows[j]
            v, ix = rj[0], rj[1]
            better = (v > gm) | ((v == gm) & (ix < gi_f))
            gm = jnp.where(better, v, gm)
            gi_f = jnp.where(better, ix, gi_f)
            bx1s = jnp.where(better, rj[2], bx1s)
            by1s = jnp.where(better, rj[3], by1s)
            bx2s = jnp.where(better, rj[4], bx2s)
            by2s = jnp.where(better, rj[5], by2s)
        valid = gm > jnp.float32(-5e29)
        gi = gi_f.astype(jnp.int32)
        gloc = gi - base
        rec = jnp.clip(gloc, 0, NSH - 1)
        plsc.store_scatter(
            outv, [jnp.zeros((L,), jnp.int32) + rec], zerov + gm,
            mask=(iota == 0) & valid & (gloc >= 0) & (gloc < NSH))
        bx1 = jnp.where(valid, zerov + bx1s, bigv)
        by1 = jnp.where(valid, zerov + by1s, bigv)
        bx2 = jnp.where(valid, zerov + bx2s, bigv)
        by2 = jnp.where(valid, zerov + by2s, bigv)
        barea = (bx2 - bx1) * (by2 - by1)
        bl = jnp.where(valid, gloc, jnp.int32(-1))
        return local_pass(bl, bx1, by1, bx2, by2, barea)

    lax.fori_loop(0, MAX_OUT, round_body, (gm_l, gi_l))
    pltpu.sync_copy(outv, outh.at[row])


@functools.partial(jax.jit)
def _nms_sc(x1, y1, x2, y2, scores):
    f = pl.kernel(
        _nms_tile_body,
        mesh=plsc.VectorSubcoreMesh(core_axis_name="c", subcore_axis_name="s"),
        out_type=[jax.ShapeDtypeStruct((B * SH, NSH), jnp.float32),
                  jax.ShapeDtypeStruct((64, L), jnp.float32)],
        compiler_params=pltpu.CompilerParams(needs_layout_passes=False),
        scratch_types=[
            pltpu.VMEM((NSH,), jnp.float32),
            pltpu.VMEM((NSH,), jnp.float32),
            pltpu.VMEM((NSH,), jnp.float32),
            pltpu.VMEM((NSH,), jnp.float32),
            pltpu.VMEM((NSH,), jnp.float32),
            pltpu.VMEM((NSH,), jnp.float32),
            pltpu.VMEM((L,), jnp.float32),
            pltpu.VMEM((L,), jnp.float32),
            pltpu.VMEM((L,), jnp.float32),
            pltpu.VMEM((L,), jnp.float32),
            pltpu.VMEM((L,), jnp.float32),
            pltpu.SemaphoreType.DMA,
            pltpu.SemaphoreType.DMA,
            pltpu.SemaphoreType.DMA,
            pltpu.SemaphoreType.DMA,
        ],
    )
    out, _mail = f(x1.reshape(B * SH, NSH), y1.reshape(B * SH, NSH),
                   x2.reshape(B * SH, NSH), y2.reshape(B * SH, NSH),
                   scores.reshape(B * SH, NSH))
    return out.reshape(B, N)


def _decode_body(x_ref, y_ref, w_ref, h_ref, ox_ref, oy_ref,
                 xmin_ref, ymin_ref, xmax_ref, ymax_ref):
    x = x_ref[...]
    y = y_ref[...]
    w = w_ref[...]
    h = h_ref[...]
    ox = ox_ref[...][None]
    oy = oy_ref[...][None]
    cx = (x + ox) * 8.0
    ws = w * 8.0
    cy = (y + oy) * 8.0
    hs = h * 8.0
    cy = 512.0 - cy
    xmin_ref[...] = cx - ws / 2.0
    ymin_ref[...] = cy - hs / 2.0
    xmax_ref[...] = cx + ws / 2.0
    ymax_ref[...] = cy + hs / 2.0


def _decode_tc(x, y, w, h, ox, oy):
    shp = jax.ShapeDtypeStruct((B, NR, NL), jnp.float32)
    return pl.pallas_call(
        _decode_body,
        out_shape=[shp, shp, shp, shp],
    )(x, y, w, h, ox, oy)


def kernel(y_pred):
    coord = y_pred[..., :4]
    conf = y_pred[..., 4]

    x = coord[..., 0].reshape(B, NR, NL)
    y = coord[..., 1].reshape(B, NR, NL)
    w = coord[..., 2].reshape(B, NR, NL)
    h = coord[..., 3].reshape(B, NR, NL)

    n = jnp.arange(N)
    ox = ((n // NBOX) % GW).astype(jnp.float32).reshape(NR, NL)
    oy = (n // (GW * NBOX)).astype(jnp.float32).reshape(NR, NL)

    xmin, ymin, xmax, ymax = _decode_tc(x, y, w, h, ox, oy)
    coords_out = jnp.stack(
        [xmin.reshape(B, N), ymin.reshape(B, N),
         xmax.reshape(B, N), ymax.reshape(B, N)], axis=-1)

    scores = conf.reshape(B, N)
    conf_nms = _nms_sc(x.reshape(B, N), y.reshape(B, N),
                       w.reshape(B, N), h.reshape(B, N), scores)

    return coords_out, conf_nms.reshape(B, N, 1)


# speedup vs baseline: 1.0661x; 1.0661x over previous
import functools

import jax
import jax.numpy as jnp
import numpy as np
from jax import lax
from jax.experimental import pallas as pl
from jax.experimental.pallas import tpu as pltpu
from jax.experimental.pallas import tpu_sc as plsc

B = 8
GH = GW = 64
NBOX = 5
N = GH * GW * NBOX
NR, NL = 160, 128
MAX_OUT = 100
IOU_T = 0.4
NEG = np.float32(-1e30)
L = 16
STEPS = N // L


SH = 4
NSH = N // SH
SSTEPS = NSH // L


def _nms_tile_body(x1h, y1h, x2h, y2h, sch, outh, mailh,
                   x1v, y1v, x2v, y2v, livev, outv, candv,
                   r0v, r1v, r2v, r3v, sem0, sem1, sem2, sem3):
    cid = lax.axis_index("c")
    sid = lax.axis_index("s")
    img = cid * 4 + sid // SH
    shard = sid % SH
    base = shard * NSH
    row = cid * 16 + sid

    pltpu.sync_copy(x1h.at[row], x1v)
    pltpu.sync_copy(y1h.at[row], y1v)
    pltpu.sync_copy(x2h.at[row], x2v)
    pltpu.sync_copy(y2h.at[row], y2v)
    pltpu.sync_copy(sch.at[row], livev)

    iota = lax.iota(jnp.int32, L)
    zeros16 = jnp.zeros((L,), jnp.float32)
    big = jnp.float32(3e38)
    bigv = jnp.full((L,), big, jnp.float32)
    zerov = jnp.zeros((L,), jnp.float32)

    @plsc.parallel_loop(0, SSTEPS, unroll=8)
    def _zero(s):
        outv[pl.ds(s * L, L)] = zeros16

    def local_pass(best_loc, bx1, by1, bx2, by2, barea):
        init = (jnp.full((L,), -3e38, jnp.float32),
                jnp.full((L,), NSH, jnp.int32))

        @plsc.parallel_loop(0, SSTEPS, unroll=16, carry=init)
        def scan(s, carry):
            rmax, ridx = carry
            sl = pl.ds(s * L, L)
            x1s = x1v[sl]
            y1s = y1v[sl]
            x2s = x2v[sl]
            y2s = y2v[sl]
            lv = livev[sl]
            iw = jnp.maximum(jnp.minimum(x2s, bx2) - jnp.maximum(x1s, bx1), 0.0)
            ih = jnp.maximum(jnp.minimum(y2s, by2) - jnp.maximum(y1s, by1), 0.0)
            inter = iw * ih
            areas = (x2s - x1s) * (y2s - y1s)
            iou = inter / jnp.maximum(areas + barea - inter, 1e-9)
            idxs = iota + s * L
            supp = (iou > IOU_T) | (idxs == best_loc)
            nl = jnp.where(supp, NEG, lv)
            livev[sl] = nl
            upd = (nl > rmax) | ((nl == rmax) & (idxs < ridx))
            rmax = jnp.where(upd, nl, rmax)
            ridx = jnp.where(upd, idxs, ridx)
            return rmax, ridx

        rmax, ridx = scan
        gm = jnp.max(rmax)
        gi = jnp.min(jnp.where(rmax == gm, ridx, jnp.int32(NSH)))
        return gm, gi

    gm_l, gi_l = local_pass(jnp.int32(-1), bigv, bigv, bigv, bigv, zerov)

    def round_body(r, carry):
        gm_l, gi_l = carry
        par = r % 2
        rtag = r.astype(jnp.float32) + 0.25
        gsplat = jnp.zeros((L,), jnp.int32) + gi_l
        cx1 = plsc.load_gather(x1v, [gsplat])
        cy1 = plsc.load_gather(y1v, [gsplat])
        cx2 = plsc.load_gather(x2v, [gsplat])
        cy2 = plsc.load_gather(y2v, [gsplat])
        cand = jnp.where(iota == 0, zerov + gm_l,
               jnp.where(iota == 1, zerov + (gi_l + base).astype(jnp.float32),
               jnp.where(iota == 2, cx1,
               jnp.where(iota == 3, cy1,
               jnp.where(iota == 4, cx2,
               jnp.where(iota == 5, cy2, zerov + rtag))))))
        candv[...] = cand
        slot = par * 32 + img * SH
        pltpu.sync_copy(candv, mailh.at[slot + shard])
        plsc.subcore_barrier()

        def read_rows():
            c0 = pltpu.make_async_copy(mailh.at[slot + 0], r0v, sem0)
            c1 = pltpu.make_async_copy(mailh.at[slot + 1], r1v, sem1)
            c2 = pltpu.make_async_copy(mailh.at[slot + 2], r2v, sem2)
            c3 = pltpu.make_async_copy(mailh.at[slot + 3], r3v, sem3)
            c0.start(); c1.start(); c2.start(); c3.start()
            c0.wait(); c1.wait(); c2.wait(); c3.wait()
            return r0v[...], r1v[...], r2v[...], r3v[...]

        def stale(rows):
            return ((rows[0][6] != rtag) | (rows[1][6] != rtag)
                    | (rows[2][6] != rtag) | (rows[3][6] != rtag))

        rows0 = read_rows()
        rows = lax.while_loop(stale, lambda _: read_rows(), rows0)
        r0 = rows[0]
        gm, gi_f = r0[0], r0[1]
        bx1s, by1s, bx2s, by2s = r0[2], r0[3], r0[4], r0[5]
        for j in range(1, SH):
            rj = rows[j]
            v, ix = rj[0], rj[1]
            better = (v > gm) | ((v == gm) & (ix < gi_f))
            gm = jnp.where(better, v, gm)
            gi_f = jnp.where(better, ix, gi_f)
            bx1s = jnp.where(better, rj[2], bx1s)
            by1s = jnp.where(better, rj[3], by1s)
            bx2s = jnp.where(better, rj[4], bx2s)
            by2s = jnp.where(better, rj[5], by2s)
        valid = gm > jnp.float32(-5e29)
        gi = gi_f.astype(jnp.int32)
        gloc = gi - base
        rec = jnp.clip(gloc, 0, NSH - 1)
        plsc.store_scatter(
            outv, [jnp.zeros((L,), jnp.int32) + rec], zerov + gm,
            mask=(iota == 0) & valid & (gloc >= 0) & (gloc < NSH))
        bx1 = jnp.where(valid, zerov + bx1s, bigv)
        by1 = jnp.where(valid, zerov + by1s, bigv)
        bx2 = jnp.where(valid, zerov + bx2s, bigv)
        by2 = jnp.where(valid, zerov + by2s, bigv)
        barea = (bx2 - bx1) * (by2 - by1)
        bl = jnp.where(valid, gloc, jnp.int32(-1))
        return local_pass(bl, bx1, by1, bx2, by2, barea)

    lax.fori_loop(0, MAX_OUT, round_body, (gm_l, gi_l))
    pltpu.sync_copy(outv, outh.at[row])


@functools.partial(jax.jit)
def _nms_sc(x1, y1, x2, y2, scores):
    f = pl.kernel(
        _nms_tile_body,
        mesh=plsc.VectorSubcoreMesh(core_axis_name="c", subcore_axis_name="s"),
        out_type=[jax.ShapeDtypeStruct((B * SH, NSH), jnp.float32),
                  jax.ShapeDtypeStruct((64, L), jnp.float32)],
        compiler_params=pltpu.CompilerParams(needs_layout_passes=False),
        scratch_types=[
            pltpu.VMEM((NSH,), jnp.float32),
            pltpu.VMEM((NSH,), jnp.float32),
            pltpu.VMEM((NSH,), jnp.float32),
            pltpu.VMEM((NSH,), jnp.float32),
            pltpu.VMEM((NSH,), jnp.float32),
            pltpu.VMEM((NSH,), jnp.float32),
            pltpu.VMEM((L,), jnp.float32),
            pltpu.VMEM((L,), jnp.float32),
            pltpu.VMEM((L,), jnp.float32),
            pltpu.VMEM((L,), jnp.float32),
            pltpu.VMEM((L,), jnp.float32),
            pltpu.SemaphoreType.DMA,
            pltpu.SemaphoreType.DMA,
            pltpu.SemaphoreType.DMA,
            pltpu.SemaphoreType.DMA,
        ],
    )
    out, _mail = f(x1.reshape(B * SH, NSH), y1.reshape(B * SH, NSH),
                   x2.reshape(B * SH, NSH), y2.reshape(B * SH, NSH),
                   scores.reshape(B * SH, NSH))
    return out.reshape(B, N)


def _decode_body(x_ref, y_ref, w_ref, h_ref, ox_ref, oy_ref,
                 xmin_ref, ymin_ref, xmax_ref, ymax_ref):
    x = x_ref[...]
    y = y_ref[...]
    w = w_ref[...]
    h = h_ref[...]
    ox = ox_ref[...][None]
    oy = oy_ref[...][None]
    cx = (x + ox) * 8.0
    ws = w * 8.0
    cy = (y + oy) * 8.0
    hs = h * 8.0
    cy = 512.0 - cy
    xmin_ref[...] = cx - ws / 2.0
    ymin_ref[...] = cy - hs / 2.0
    xmax_ref[...] = cx + ws / 2.0
    ymax_ref[...] = cy + hs / 2.0


def _decode_tc(x, y, w, h, ox, oy):
    shp = jax.ShapeDtypeStruct((B, NR, NL), jnp.float32)
    return pl.pallas_call(
        _decode_body,
        out_shape=[shp, shp, shp, shp],
    )(x, y, w, h, ox, oy)


def kernel(y_pred):
    coord = y_pred[..., :4]
    conf = y_pred[..., 4]

    x = coord[..., 0].reshape(B, NR, NL)
    y = coord[..., 1].reshape(B, NR, NL)
    w = coord[..., 2].reshape(B, NR, NL)
    h = coord[..., 3].reshape(B, NR, NL)

    n = jnp.arange(N)
    ox = ((n // NBOX) % GW).astype(jnp.float32).reshape(NR, NL)
    oy = (n // (GW * NBOX)).astype(jnp.float32).reshape(NR, NL)

    xmin, ymin, xmax, ymax = _decode_tc(x, y, w, h, ox, oy)
    coords_out = jnp.stack(
        [xmin.reshape(B, N), ymin.reshape(B, N),
         xmax.reshape(B, N), ymax.reshape(B, N)], axis=-1)

    scores = conf.reshape(B, N)
    conf_nms = _nms_sc(x.reshape(B, N), y.reshape(B, N),
                       w.reshape(B, N), h.reshape(B, N), scores)

    return coords_out, conf_nms.reshape(B, N, 1)


# speedup vs baseline: 1.0878x; 1.0203x over previous
import functools

import jax
import jax.numpy as jnp
import numpy as np
from jax import lax
from jax.experimental import pallas as pl
from jax.experimental.pallas import tpu as pltpu
from jax.experimental.pallas import tpu_sc as plsc

B = 8
GH = GW = 64
NBOX = 5
N = GH * GW * NBOX
NR, NL = 160, 128
MAX_OUT = 100
IOU_T = 0.4
NEG = np.float32(-1e30)
L = 16
STEPS = N // L


SH = 4
NSH = N // SH
SSTEPS = NSH // L


def _nms_tile_body(x1h, y1h, x2h, y2h, sch, outh, mailh,
                   x1v, y1v, x2v, y2v, livev, outv, candv,
                   r0v, r1v, r2v, r3v, sem0, sem1, sem2, sem3):
    cid = lax.axis_index("c")
    sid = lax.axis_index("s")
    img = cid * 4 + sid // SH
    shard = sid % SH
    base = shard * NSH
    row = cid * 16 + sid

    pltpu.sync_copy(x1h.at[row], x1v)
    pltpu.sync_copy(y1h.at[row], y1v)
    pltpu.sync_copy(x2h.at[row], x2v)
    pltpu.sync_copy(y2h.at[row], y2v)
    pltpu.sync_copy(sch.at[row], livev)

    iota = lax.iota(jnp.int32, L)
    zeros16 = jnp.zeros((L,), jnp.float32)
    big = jnp.float32(3e38)
    bigv = jnp.full((L,), big, jnp.float32)
    zerov = jnp.zeros((L,), jnp.float32)

    @plsc.parallel_loop(0, SSTEPS, unroll=8)
    def _zero(s):
        outv[pl.ds(s * L, L)] = zeros16

    def local_pass(best_loc, bx1, by1, bx2, by2, barea):
        init = (jnp.full((L,), -3e38, jnp.float32),
                jnp.full((L,), NSH, jnp.int32))

        @plsc.parallel_loop(0, SSTEPS, unroll=32, carry=init)
        def scan(s, carry):
            rmax, ridx = carry
            sl = pl.ds(s * L, L)
            x1s = x1v[sl]
            y1s = y1v[sl]
            x2s = x2v[sl]
            y2s = y2v[sl]
            lv = livev[sl]
            iw = jnp.maximum(jnp.minimum(x2s, bx2) - jnp.maximum(x1s, bx1), 0.0)
            ih = jnp.maximum(jnp.minimum(y2s, by2) - jnp.maximum(y1s, by1), 0.0)
            inter = iw * ih
            areas = (x2s - x1s) * (y2s - y1s)
            iou = inter / jnp.maximum(areas + barea - inter, 1e-9)
            idxs = iota + s * L
            supp = (iou > IOU_T) | (idxs == best_loc)
            nl = jnp.where(supp, NEG, lv)
            livev[sl] = nl
            upd = (nl > rmax) | ((nl == rmax) & (idxs < ridx))
            rmax = jnp.where(upd, nl, rmax)
            ridx = jnp.where(upd, idxs, ridx)
            return rmax, ridx

        rmax, ridx = scan
        gm = jnp.max(rmax)
        gi = jnp.min(jnp.where(rmax == gm, ridx, jnp.int32(NSH)))
        return gm, gi

    gm_l, gi_l = local_pass(jnp.int32(-1), bigv, bigv, bigv, bigv, zerov)

    def round_body(r, carry):
        gm_l, gi_l = carry
        par = r % 2
        rtag = r.astype(jnp.float32) + 0.25
        gsplat = jnp.zeros((L,), jnp.int32) + gi_l
        cx1 = plsc.load_gather(x1v, [gsplat])
        cy1 = plsc.load_gather(y1v, [gsplat])
        cx2 = plsc.load_gather(x2v, [gsplat])
        cy2 = plsc.load_gather(y2v, [gsplat])
        cand = jnp.where(iota == 0, zerov + gm_l,
               jnp.where(iota == 1, zerov + (gi_l + base).astype(jnp.float32),
               jnp.where(iota == 2, cx1,
               jnp.where(iota == 3, cy1,
               jnp.where(iota == 4, cx2,
               jnp.where(iota == 5, cy2, zerov + rtag))))))
        candv[...] = cand
        slot = par * 32 + img * SH
        pltpu.sync_copy(candv, mailh.at[slot + shard])
        plsc.subcore_barrier()

        def read_rows():
            c0 = pltpu.make_async_copy(mailh.at[slot + 0], r0v, sem0)
            c1 = pltpu.make_async_copy(mailh.at[slot + 1], r1v, sem1)
            c2 = pltpu.make_async_copy(mailh.at[slot + 2], r2v, sem2)
            c3 = pltpu.make_async_copy(mailh.at[slot + 3], r3v, sem3)
            c0.start(); c1.start(); c2.start(); c3.start()
            c0.wait(); c1.wait(); c2.wait(); c3.wait()
            return r0v[...], r1v[...], r2v[...], r3v[...]

        def stale(rows):
            return ((rows[0][6] != rtag) | (rows[1][6] != rtag)
                    | (rows[2][6] != rtag) | (rows[3][6] != rtag))

        rows0 = read_rows()
        rows = lax.while_loop(stale, lambda _: read_rows(), rows0)
        r0 = rows[0]
        gm, gi_f = r0[0], r0[1]
        bx1s, by1s, bx2s, by2s = r0[2], r0[3], r0[4], r0[5]
        for j in range(1, SH):
            rj = rows[j]
            v, ix = rj[0], rj[1]
            better = (v > gm) | ((v == gm) & (ix < gi_f))
            gm = jnp.where(better, v, gm)
            gi_f = jnp.where(better, ix, gi_f)
            bx1s = jnp.where(better, rj[2], bx1s)
            by1s = jnp.where(better, rj[3], by1s)
            bx2s = jnp.where(better, rj[4], bx2s)
            by2s = jnp.where(better, rj[5], by2s)
        valid = gm > jnp.float32(-5e29)
        gi = gi_f.astype(jnp.int32)
        gloc = gi - base
        rec = jnp.clip(gloc, 0, NSH - 1)
        plsc.store_scatter(
            outv, [jnp.zeros((L,), jnp.int32) + rec], zerov + gm,
            mask=(iota == 0) & valid & (gloc >= 0) & (gloc < NSH))
        bx1 = jnp.where(valid, zerov + bx1s, bigv)
        by1 = jnp.where(valid, zerov + by1s, bigv)
        bx2 = jnp.where(valid, zerov + bx2s, bigv)
        by2 = jnp.where(valid, zerov + by2s, bigv)
        barea = (bx2 - bx1) * (by2 - by1)
        bl = jnp.where(valid, gloc, jnp.int32(-1))
        return local_pass(bl, bx1, by1, bx2, by2, barea)

    lax.fori_loop(0, MAX_OUT, round_body, (gm_l, gi_l))
    pltpu.sync_copy(outv, outh.at[row])


@functools.partial(jax.jit)
def _nms_sc(x1, y1, x2, y2, scores):
    f = pl.kernel(
        _nms_tile_body,
        mesh=plsc.VectorSubcoreMesh(core_axis_name="c", subcore_axis_name="s"),
        out_type=[jax.ShapeDtypeStruct((B * SH, NSH), jnp.float32),
                  jax.ShapeDtypeStruct((64, L), jnp.float32)],
        compiler_params=pltpu.CompilerParams(needs_layout_passes=False),
        scratch_types=[
            pltpu.VMEM((NSH,), jnp.float32),
            pltpu.VMEM((NSH,), jnp.float32),
            pltpu.VMEM((NSH,), jnp.float32),
            pltpu.VMEM((NSH,), jnp.float32),
            pltpu.VMEM((NSH,), jnp.float32),
            pltpu.VMEM((NSH,), jnp.float32),
            pltpu.VMEM((L,), jnp.float32),
            pltpu.VMEM((L,), jnp.float32),
            pltpu.VMEM((L,), jnp.float32),
            pltpu.VMEM((L,), jnp.float32),
            pltpu.VMEM((L,), jnp.float32),
            pltpu.SemaphoreType.DMA,
            pltpu.SemaphoreType.DMA,
            pltpu.SemaphoreType.DMA,
            pltpu.SemaphoreType.DMA,
        ],
    )
    out, _mail = f(x1.reshape(B * SH, NSH), y1.reshape(B * SH, NSH),
                   x2.reshape(B * SH, NSH), y2.reshape(B * SH, NSH),
                   scores.reshape(B * SH, NSH))
    return out.reshape(B, N)


def _decode_body(x_ref, y_ref, w_ref, h_ref, ox_ref, oy_ref,
                 xmin_ref, ymin_ref, xmax_ref, ymax_ref):
    x = x_ref[...]
    y = y_ref[...]
    w = w_ref[...]
    h = h_ref[...]
    ox = ox_ref[...][None]
    oy = oy_ref[...][None]
    cx = (x + ox) * 8.0
    ws = w * 8.0
    cy = (y + oy) * 8.0
    hs = h * 8.0
    cy = 512.0 - cy
    xmin_ref[...] = cx - ws / 2.0
    ymin_ref[...] = cy - hs / 2.0
    xmax_ref[...] = cx + ws / 2.0
    ymax_ref[...] = cy + hs / 2.0


def _decode_tc(x, y, w, h, ox, oy):
    shp = jax.ShapeDtypeStruct((B, NR, NL), jnp.float32)
    return pl.pallas_call(
        _decode_body,
        out_shape=[shp, shp, shp, shp],
    )(x, y, w, h, ox, oy)


def kernel(y_pred):
    coord = y_pred[..., :4]
    conf = y_pred[..., 4]

    x = coord[..., 0].reshape(B, NR, NL)
    y = coord[..., 1].reshape(B, NR, NL)
    w = coord[..., 2].reshape(B, NR, NL)
    h = coord[..., 3].reshape(B, NR, NL)

    n = jnp.arange(N)
    ox = ((n // NBOX) % GW).astype(jnp.float32).reshape(NR, NL)
    oy = (n // (GW * NBOX)).astype(jnp.float32).reshape(NR, NL)

    xmin, ymin, xmax, ymax = _decode_tc(x, y, w, h, ox, oy)
    coords_out = jnp.stack(
        [xmin.reshape(B, N), ymin.reshape(B, N),
         xmax.reshape(B, N), ymax.reshape(B, N)], axis=-1)

    scores = conf.reshape(B, N)
    conf_nms = _nms_sc(x.reshape(B, N), y.reshape(B, N),
                       w.reshape(B, N), h.reshape(B, N), scores)

    return coords_out, conf_nms.reshape(B, N, 1)


# speedup vs baseline: 1.0898x; 1.0018x over previous
import functools

import jax
import jax.numpy as jnp
import numpy as np
from jax import lax
from jax.experimental import pallas as pl
from jax.experimental.pallas import tpu as pltpu
from jax.experimental.pallas import tpu_sc as plsc

B = 8
GH = GW = 64
NBOX = 5
N = GH * GW * NBOX
NR, NL = 160, 128
MAX_OUT = 100
IOU_T = 0.4
NEG = np.float32(-1e30)
L = 16
STEPS = N // L


SH = 4
NSH = N // SH
SSTEPS = NSH // L


def _nms_tile_body(x1h, y1h, x2h, y2h, sch, outh, mailh,
                   x1v, y1v, x2v, y2v, livev, outv, candv,
                   r0v, r1v, r2v, r3v, sem0, sem1, sem2, sem3):
    cid = lax.axis_index("c")
    sid = lax.axis_index("s")
    img = cid * 4 + sid // SH
    shard = sid % SH
    base = shard * NSH
    row = cid * 16 + sid

    pltpu.sync_copy(x1h.at[row], x1v)
    pltpu.sync_copy(y1h.at[row], y1v)
    pltpu.sync_copy(x2h.at[row], x2v)
    pltpu.sync_copy(y2h.at[row], y2v)
    pltpu.sync_copy(sch.at[row], livev)

    iota = lax.iota(jnp.int32, L)
    zeros16 = jnp.zeros((L,), jnp.float32)
    big = jnp.float32(3e38)
    bigv = jnp.full((L,), big, jnp.float32)
    zerov = jnp.zeros((L,), jnp.float32)

    @plsc.parallel_loop(0, SSTEPS, unroll=8)
    def _zero(s):
        outv[pl.ds(s * L, L)] = zeros16

    def local_pass(best_loc, bx1, by1, bx2, by2, barea):
        init = (jnp.full((L,), -3e38, jnp.float32),
                jnp.full((L,), NSH, jnp.int32))

        @plsc.parallel_loop(0, NSH, step=L, unroll=32, carry=init)
        def scan(s, carry):
            rmax, ridx = carry
            sl = pl.ds(s, L)
            x1s = x1v[sl]
            y1s = y1v[sl]
            x2s = x2v[sl]
            y2s = y2v[sl]
            lv = livev[sl]
            iw = jnp.maximum(jnp.minimum(x2s, bx2) - jnp.maximum(x1s, bx1), 0.0)
            ih = jnp.maximum(jnp.minimum(y2s, by2) - jnp.maximum(y1s, by1), 0.0)
            inter = iw * ih
            areas = (x2s - x1s) * (y2s - y1s)
            iou = inter / jnp.maximum(areas + barea - inter, 1e-9)
            idxs = iota + s
            supp = (iou > IOU_T) | (idxs == best_loc)
            nl = jnp.where(supp, NEG, lv)
            livev[sl] = nl
            upd = (nl > rmax) | ((nl == rmax) & (idxs < ridx))
            rmax = jnp.where(upd, nl, rmax)
            ridx = jnp.where(upd, idxs, ridx)
            return rmax, ridx

        rmax, ridx = scan
        gm = jnp.max(rmax)
        gi = jnp.min(jnp.where(rmax == gm, ridx, jnp.int32(NSH)))
        return gm, gi

    gm_l, gi_l = local_pass(jnp.int32(-1), bigv, bigv, bigv, bigv, zerov)

    def round_body(r, carry):
        gm_l, gi_l = carry
        par = r % 2
        rtag = r.astype(jnp.float32) + 0.25
        gsplat = jnp.zeros((L,), jnp.int32) + gi_l
        cx1 = plsc.load_gather(x1v, [gsplat])
        cy1 = plsc.load_gather(y1v, [gsplat])
        cx2 = plsc.load_gather(x2v, [gsplat])
        cy2 = plsc.load_gather(y2v, [gsplat])
        cand = jnp.where(iota == 0, zerov + gm_l,
               jnp.where(iota == 1, zerov + (gi_l + base).astype(jnp.float32),
               jnp.where(iota == 2, cx1,
               jnp.where(iota == 3, cy1,
               jnp.where(iota == 4, cx2,
               jnp.where(iota == 5, cy2, zerov + rtag))))))
        candv[...] = cand
        slot = par * 32 + img * SH
        pltpu.sync_copy(candv, mailh.at[slot + shard])
        plsc.subcore_barrier()

        def read_rows():
            c0 = pltpu.make_async_copy(mailh.at[slot + 0], r0v, sem0)
            c1 = pltpu.make_async_copy(mailh.at[slot + 1], r1v, sem1)
            c2 = pltpu.make_async_copy(mailh.at[slot + 2], r2v, sem2)
            c3 = pltpu.make_async_copy(mailh.at[slot + 3], r3v, sem3)
            c0.start(); c1.start(); c2.start(); c3.start()
            c0.wait(); c1.wait(); c2.wait(); c3.wait()
            return r0v[...], r1v[...], r2v[...], r3v[...]

        def stale(rows):
            return ((rows[0][6] != rtag) | (rows[1][6] != rtag)
                    | (rows[2][6] != rtag) | (rows[3][6] != rtag))

        rows0 = read_rows()
        rows = lax.while_loop(stale, lambda _: read_rows(), rows0)
        r0 = rows[0]
        gm, gi_f = r0[0], r0[1]
        bx1s, by1s, bx2s, by2s = r0[2], r0[3], r0[4], r0[5]
        for j in range(1, SH):
            rj = rows[j]
            v, ix = rj[0], rj[1]
            better = (v > gm) | ((v == gm) & (ix < gi_f))
            gm = jnp.where(better, v, gm)
            gi_f = jnp.where(better, ix, gi_f)
            bx1s = jnp.where(better, rj[2], bx1s)
            by1s = jnp.where(better, rj[3], by1s)
            bx2s = jnp.where(better, rj[4], bx2s)
            by2s = jnp.where(better, rj[5], by2s)
        valid = gm > jnp.float32(-5e29)
        gi = gi_f.astype(jnp.int32)
        gloc = gi - base
        rec = jnp.clip(gloc, 0, NSH - 1)
        plsc.store_scatter(
            outv, [jnp.zeros((L,), jnp.int32) + rec], zerov + gm,
            mask=(iota == 0) & valid & (gloc >= 0) & (gloc < NSH))
        bx1 = jnp.where(valid, zerov + bx1s, bigv)
        by1 = jnp.where(valid, zerov + by1s, bigv)
        bx2 = jnp.where(valid, zerov + bx2s, bigv)
        by2 = jnp.where(valid, zerov + by2s, bigv)
        barea = (bx2 - bx1) * (by2 - by1)
        bl = jnp.where(valid, gloc, jnp.int32(-1))
        return local_pass(bl, bx1, by1, bx2, by2, barea)

    lax.fori_loop(0, MAX_OUT, round_body, (gm_l, gi_l))
    pltpu.sync_copy(outv, outh.at[row])


@functools.partial(jax.jit)
def _nms_sc(x1, y1, x2, y2, scores):
    f = pl.kernel(
        _nms_tile_body,
        mesh=plsc.VectorSubcoreMesh(core_axis_name="c", subcore_axis_name="s"),
        out_type=[jax.ShapeDtypeStruct((B * SH, NSH), jnp.float32),
                  jax.ShapeDtypeStruct((64, L), jnp.float32)],
        compiler_params=pltpu.CompilerParams(needs_layout_passes=False),
        scratch_types=[
            pltpu.VMEM((NSH,), jnp.float32),
            pltpu.VMEM((NSH,), jnp.float32),
            pltpu.VMEM((NSH,), jnp.float32),
            pltpu.VMEM((NSH,), jnp.float32),
            pltpu.VMEM((NSH,), jnp.float32),
            pltpu.VMEM((NSH,), jnp.float32),
            pltpu.VMEM((L,), jnp.float32),
            pltpu.VMEM((L,), jnp.float32),
            pltpu.VMEM((L,), jnp.float32),
            pltpu.VMEM((L,), jnp.float32),
            pltpu.VMEM((L,), jnp.float32),
            pltpu.SemaphoreType.DMA,
            pltpu.SemaphoreType.DMA,
            pltpu.SemaphoreType.DMA,
            pltpu.SemaphoreType.DMA,
        ],
    )
    out, _mail = f(x1.reshape(B * SH, NSH), y1.reshape(B * SH, NSH),
                   x2.reshape(B * SH, NSH), y2.reshape(B * SH, NSH),
                   scores.reshape(B * SH, NSH))
    return out.reshape(B, N)


def _decode_body(x_ref, y_ref, w_ref, h_ref, ox_ref, oy_ref,
                 xmin_ref, ymin_ref, xmax_ref, ymax_ref):
    x = x_ref[...]
    y = y_ref[...]
    w = w_ref[...]
    h = h_ref[...]
    ox = ox_ref[...][None]
    oy = oy_ref[...][None]
    cx = (x + ox) * 8.0
    ws = w * 8.0
    cy = (y + oy) * 8.0
    hs = h * 8.0
    cy = 512.0 - cy
    xmin_ref[...] = cx - ws / 2.0
    ymin_ref[...] = cy - hs / 2.0
    xmax_ref[...] = cx + ws / 2.0
    ymax_ref[...] = cy + hs / 2.0


def _decode_tc(x, y, w, h, ox, oy):
    shp = jax.ShapeDtypeStruct((B, NR, NL), jnp.float32)
    return pl.pallas_call(
        _decode_body,
        out_shape=[shp, shp, shp, shp],
    )(x, y, w, h, ox, oy)


def kernel(y_pred):
    coord = y_pred[..., :4]
    conf = y_pred[..., 4]

    x = coord[..., 0].reshape(B, NR, NL)
    y = coord[..., 1].reshape(B, NR, NL)
    w = coord[..., 2].reshape(B, NR, NL)
    h = coord[..., 3].reshape(B, NR, NL)

    n = jnp.arange(N)
    ox = ((n // NBOX) % GW).astype(jnp.float32).reshape(NR, NL)
    oy = (n // (GW * NBOX)).astype(jnp.float32).reshape(NR, NL)

    xmin, ymin, xmax, ymax = _decode_tc(x, y, w, h, ox, oy)
    coords_out = jnp.stack(
        [xmin.reshape(B, N), ymin.reshape(B, N),
         xmax.reshape(B, N), ymax.reshape(B, N)], axis=-1)

    scores = conf.reshape(B, N)
    conf_nms = _nms_sc(x.reshape(B, N), y.reshape(B, N),
                       w.reshape(B, N), h.reshape(B, N), scores)

    return coords_out, conf_nms.reshape(B, N, 1)
